# baseline (speedup 1.0000x reference)
import functools

import jax
import jax.numpy as jnp
from jax.experimental import pallas as pl
from jax.experimental.pallas import tpu as pltpu

N_RAW = 5000
N = 5120
T = 512
NT = N // T
CONF_C = 0.25
CONF_P = 0.001
IOU_THR = 0.45
MAX_WH = 4096.0
CAP_C = 1000.0


def _prep_kernel(x_ref, out_ref, *, conf_thres):
    x = x_ref[0]
    obj = x[:, 4:5]
    scores = x[:, 5:85] * obj
    conf = jnp.max(scores, axis=1, keepdims=True)
    ji = jax.lax.broadcasted_iota(jnp.int32, (N, 80), 1).astype(jnp.float32)
    cls = jnp.min(jnp.where(scores == conf, ji, 1e9), axis=1, keepdims=True)
    valid = ((obj > conf_thres) & (conf > conf_thres)).astype(jnp.float32)
    w2 = x[:, 2:3] * 0.5
    h2 = x[:, 3:4] * 0.5
    x1 = x[:, 0:1] - w2
    y1 = x[:, 1:2] - h2
    x2 = x[:, 0:1] + w2
    y2 = x[:, 1:2] + h2
    out_ref[0] = jnp.concatenate(
        [x1, y1, x2, y2, cls, valid, conf, jnp.zeros_like(conf)], axis=1)


def _tile_coords(rb_ref, cbt_ref, r0, c0, offset):
    roff = rb_ref[0, pl.ds(r0, T), 4:5] * MAX_WH if offset else 0.0
    coff = cbt_ref[0, 4:5, pl.ds(c0, T)] * MAX_WH if offset else 0.0
    rx1 = rb_ref[0, pl.ds(r0, T), 0:1] + roff
    ry1 = rb_ref[0, pl.ds(r0, T), 1:2] + roff
    rx2 = rb_ref[0, pl.ds(r0, T), 2:3] + roff
    ry2 = rb_ref[0, pl.ds(r0, T), 3:4] + roff
    cx1 = cbt_ref[0, 0:1, pl.ds(c0, T)] + coff
    cy1 = cbt_ref[0, 1:2, pl.ds(c0, T)] + coff
    cx2 = cbt_ref[0, 2:3, pl.ds(c0, T)] + coff
    cy2 = cbt_ref[0, 3:4, pl.ds(c0, T)] + coff
    iw = jnp.maximum(jnp.minimum(rx2, cx2) - jnp.maximum(rx1, cx1), 0.0)
    ih = jnp.maximum(jnp.minimum(ry2, cy2) - jnp.maximum(ry1, cy1), 0.0)
    inter = iw * ih
    ra = (rx2 - rx1) * (ry2 - ry1)
    ca = (cx2 - cx1) * (cy2 - cy1)
    return inter, ra, ca


def _iou_block(rb_ref, cbt_ref, r0, c0, offset):
    inter, ra, ca = _tile_coords(rb_ref, cbt_ref, r0, c0, offset)
    return inter / (ra + ca - inter)


def _sup_block(rb_ref, cbt_ref, r0, c0):
    inter, ra, ca = _tile_coords(rb_ref, cbt_ref, r0, c0, True)
    tra, tca = IOU_THR * ra, IOU_THR * ca
    za, zc = ra == 0.0, ca == 0.0
    sup = (inter * (1.0 + IOU_THR) > tra + tca) | (za & zc)
    return sup.astype(jnp.float32)


def _rowdot(k, m):
    return jnp.dot(k, m, preferred_element_type=jnp.float32)


def _nms(b_ref, bt_ref, v_ref, keep_ref, sm_ref, cap):
    keep_ref[...] = jnp.zeros((1, N), jnp.float32)
    sm_ref[0] = 0.0

    def tile_body(t, carry):
        c0 = t * T

        @pl.when((sm_ref[0] < cap) if cap is not None else (t >= 0))
        def _():
            v = v_ref[0, 0:1, pl.ds(c0, T)]

            def cross_body(u, sup):
                ku = keep_ref[0:1, pl.ds(u * T, T)]
                return sup + _rowdot(ku, _sup_block(b_ref, bt_ref, u * T, c0))

            sup = jax.lax.fori_loop(0, t, cross_body,
                                    jnp.zeros((1, T), jnp.float32))
            v2 = jnp.where(sup == 0.0, v, 0.0)

            ii = jax.lax.broadcasted_iota(jnp.int32, (T, T), 0)
            jj = jax.lax.broadcasted_iota(jnp.int32, (T, T), 1)
            s_mat = jnp.where(ii < jj, _sup_block(b_ref, bt_ref, c0, c0), 0.0)

            def cond(st):
                return st[1]

            def body(st):
                k, _ = st
                k2 = jnp.where(_rowdot(k, s_mat) == 0.0, v2, 0.0)
                return k2, jnp.any(k2 != k)

            k, _ = jax.lax.while_loop(cond, body, (v2, jnp.bool_(True)))
            if cap is not None:
                u_incl = (ii <= jj).astype(jnp.float32)
                pref = _rowdot(k, u_incl)
                k = jnp.where(sm_ref[0] + pref <= cap, k, 0.0)
            keep_ref[0:1, pl.ds(c0, T)] = k
            sm_ref[0] += jnp.sum(k)

        return carry

    jax.lax.fori_loop(0, NT, tile_body, 0)


def _main_kernel(cb_ref, cbt_ref, cv_ref, pb_ref, pbt_ref, pv_ref,
                 tot_ref, cnt_ref, kc_ref, kp_ref, sm_ref):
    _nms(cb_ref, cbt_ref, cv_ref, kc_ref, sm_ref, CAP_C)
    _nms(pb_ref, pbt_ref, pv_ref, kp_ref, sm_ref, None)

    sm_ref[1] = 0.0

    def clean_body(jc, carry):
        kc = kc_ref[0:1, pl.ds(jc * T, T)]

        @pl.when(jnp.sum(kc) > 0.0)
        def _():
            rcls = cb_ref[0, pl.ds(jc * T, T), 4:5]

            def patch_body(ip, vmax):
                kp = kp_ref[0:1, pl.ds(ip * T, T)]
                ccls = pbt_ref[0, 4:5, pl.ds(ip * T, T)]
                blk = _iou_block(cb_ref, pbt_ref, jc * T, ip * T, False)
                blk = jnp.where((kp > 0.0) & (rcls == ccls), blk, 0.0)
                return jnp.maximum(vmax, jnp.max(blk, axis=1, keepdims=True))

            vmax = jax.lax.fori_loop(0, NT, patch_body,
                                     jnp.zeros((T, 1), jnp.float32))
            sm_ref[1] += _rowdot(kc, vmax)[0, 0]

        return carry

    jax.lax.fori_loop(0, NT, clean_body, 0)
    cnt = jnp.sum(kc_ref[...])
    tot_ref[0] = jnp.full((1, 128), sm_ref[1], jnp.float32)
    cnt_ref[0] = jnp.full((1, 128), cnt, jnp.float32)


def _prep(x, conf_thres):
    b = x.shape[0]
    kfn = functools.partial(_prep_kernel, conf_thres=conf_thres)
    return pl.pallas_call(
        kfn,
        grid=(b,),
        in_specs=[pl.BlockSpec((1, N, 85), lambda i: (i, 0, 0))],
        out_specs=pl.BlockSpec((1, N, 8), lambda i: (i, 0, 0)),
        out_shape=jax.ShapeDtypeStruct((b, N, 8), jnp.float32),
    )(x)


def _order_gather(out):
    key = jnp.where(out[:, :, 5] > 0, -out[:, :, 6], jnp.inf)
    order = jnp.argsort(key, axis=1, stable=True)
    srt = jnp.take_along_axis(out, order[:, :, None], axis=1)
    bt = jnp.transpose(srt, (0, 2, 1))
    return srt[:, :, 0:5], bt[:, 0:5, :], bt[:, 5:6, :]


def kernel(output_clean, output_patch):
    x = jnp.concatenate([output_clean, output_patch], axis=0)
    x = jnp.pad(x, ((0, 0), (0, N - N_RAW), (0, 0)))
    b = output_clean.shape[0]

    cb, cbt, cv = _order_gather(_prep(x[:b], CONF_C))
    pb, pbt, pv = _order_gather(_prep(x[b:], CONF_P))

    spec_b = pl.BlockSpec((1, N, 5), lambda i: (i, 0, 0))
    spec_bt = pl.BlockSpec((1, 5, N), lambda i: (i, 0, 0))
    spec_v = pl.BlockSpec((1, 1, N), lambda i: (i, 0, 0))
    spec_o = pl.BlockSpec((1, 1, 128), lambda i: (i, 0, 0))
    tot, cnt = pl.pallas_call(
        _main_kernel,
        grid=(b,),
        in_specs=[spec_b, spec_bt, spec_v, spec_b, spec_bt, spec_v],
        out_specs=[spec_o, spec_o],
        out_shape=[
            jax.ShapeDtypeStruct((b, 1, 128), jnp.float32),
            jax.ShapeDtypeStruct((b, 1, 128), jnp.float32),
        ],
        scratch_shapes=[
            pltpu.VMEM((1, N), jnp.float32),
            pltpu.VMEM((1, N), jnp.float32),
            pltpu.SMEM((2,), jnp.float32),
        ],
    )(cb, cbt, cv, pb, pbt, pv)

    tsum = jnp.sum(tot[:, 0, 0])
    csum = jnp.sum(cnt[:, 0, 0])
    return jnp.where(csum > 0,
                     jnp.float32(1.0) - tsum / jnp.maximum(csum, 1.0),
                     jnp.float32(1.0))


# speedup vs baseline: 1.0830x; 1.0830x over previous
import functools

import jax
import jax.numpy as jnp
from jax.experimental import pallas as pl
from jax.experimental.pallas import tpu as pltpu

N_RAW = 5000
N = 5120
T = 512
NT = N // T
CONF_C = 0.25
CONF_P = 0.001
IOU_THR = 0.45
MAX_WH = 4096.0
CAP_C = 1000.0


def _prep_kernel(x_ref, conf_ref, valid_ref, boxt_ref, *, conf_thres):
    xt = x_ref[0]
    obj = xt[4:5, :]
    scores = xt[5:85, :] * obj
    conf = jnp.max(scores, axis=0, keepdims=True)
    ji = jax.lax.broadcasted_iota(jnp.int32, (80, N), 0).astype(jnp.float32)
    cls = jnp.min(jnp.where(scores == conf, ji, 1e9), axis=0, keepdims=True)
    valid = (obj > conf_thres) & (conf > conf_thres)
    xc, yc = xt[0:1, :], xt[1:2, :]
    w2, h2 = xt[2:3, :] * 0.5, xt[3:4, :] * 0.5
    x1 = xc - w2
    y1 = yc - h2
    x2 = xc + w2
    y2 = yc + h2
    conf_ref[0] = conf
    valid_ref[0] = valid.astype(jnp.float32)
    boxt_ref[0] = jnp.concatenate([x1, y1, x2, y2, cls], axis=0)


def _tile_coords(rb_ref, cbt_ref, r0, c0, offset):
    roff = rb_ref[0, pl.ds(r0, T), 4:5] * MAX_WH if offset else 0.0
    coff = cbt_ref[0, 4:5, pl.ds(c0, T)] * MAX_WH if offset else 0.0
    rx1 = rb_ref[0, pl.ds(r0, T), 0:1] + roff
    ry1 = rb_ref[0, pl.ds(r0, T), 1:2] + roff
    rx2 = rb_ref[0, pl.ds(r0, T), 2:3] + roff
    ry2 = rb_ref[0, pl.ds(r0, T), 3:4] + roff
    cx1 = cbt_ref[0, 0:1, pl.ds(c0, T)] + coff
    cy1 = cbt_ref[0, 1:2, pl.ds(c0, T)] + coff
    cx2 = cbt_ref[0, 2:3, pl.ds(c0, T)] + coff
    cy2 = cbt_ref[0, 3:4, pl.ds(c0, T)] + coff
    iw = jnp.maximum(jnp.minimum(rx2, cx2) - jnp.maximum(rx1, cx1), 0.0)
    ih = jnp.maximum(jnp.minimum(ry2, cy2) - jnp.maximum(ry1, cy1), 0.0)
    inter = iw * ih
    ra = (rx2 - rx1) * (ry2 - ry1)
    ca = (cx2 - cx1) * (cy2 - cy1)
    return inter, ra, ca


def _iou_block(rb_ref, cbt_ref, r0, c0, offset):
    inter, ra, ca = _tile_coords(rb_ref, cbt_ref, r0, c0, offset)
    return inter / (ra + ca - inter)


def _sup_block(rb_ref, cbt_ref, r0, c0):
    inter, ra, ca = _tile_coords(rb_ref, cbt_ref, r0, c0, True)
    tra, tca = IOU_THR * ra, IOU_THR * ca
    za, zc = ra == 0.0, ca == 0.0
    sup = (inter * (1.0 + IOU_THR) > tra + tca) | (za & zc)
    return sup.astype(jnp.float32)


def _rowdot(k, m):
    return jnp.dot(k, m, preferred_element_type=jnp.float32)


def _nms(b_ref, bt_ref, v_ref, keep_ref, sm_ref, cap):
    keep_ref[...] = jnp.zeros((1, N), jnp.float32)
    sm_ref[0] = 0.0

    def tile_body(t, carry):
        c0 = t * T

        @pl.when((sm_ref[0] < cap) if cap is not None else (t >= 0))
        def _():
            v = v_ref[0, 0:1, pl.ds(c0, T)]

            def cross_body(u, sup):
                ku = keep_ref[0:1, pl.ds(u * T, T)]
                return sup + _rowdot(ku, _sup_block(b_ref, bt_ref, u * T, c0))

            sup = jax.lax.fori_loop(0, t, cross_body,
                                    jnp.zeros((1, T), jnp.float32))
            v2 = jnp.where(sup == 0.0, v, 0.0)

            ii = jax.lax.broadcasted_iota(jnp.int32, (T, T), 0)
            jj = jax.lax.broadcasted_iota(jnp.int32, (T, T), 1)
            s_mat = jnp.where(ii < jj, _sup_block(b_ref, bt_ref, c0, c0), 0.0)

            def cond(st):
                return st[1]

            def body(st):
                k, _ = st
                k2 = jnp.where(_rowdot(k, s_mat) == 0.0, v2, 0.0)
                return k2, jnp.any(k2 != k)

            k, _ = jax.lax.while_loop(cond, body, (v2, jnp.bool_(True)))
            if cap is not None:
                u_incl = (ii <= jj).astype(jnp.float32)
                pref = _rowdot(k, u_incl)
                k = jnp.where(sm_ref[0] + pref <= cap, k, 0.0)
            keep_ref[0:1, pl.ds(c0, T)] = k
            sm_ref[0] += jnp.sum(k)

        return carry

    jax.lax.fori_loop(0, NT, tile_body, 0)


def _main_kernel(cb_ref, cbt_ref, cv_ref, pb_ref, pbt_ref, pv_ref,
                 tot_ref, cnt_ref, kc_ref, kp_ref, sm_ref):
    _nms(cb_ref, cbt_ref, cv_ref, kc_ref, sm_ref, CAP_C)
    _nms(pb_ref, pbt_ref, pv_ref, kp_ref, sm_ref, None)

    sm_ref[1] = 0.0

    def clean_body(jc, carry):
        kc = kc_ref[0:1, pl.ds(jc * T, T)]

        @pl.when(jnp.sum(kc) > 0.0)
        def _():
            rcls = cb_ref[0, pl.ds(jc * T, T), 4:5]

            def patch_body(ip, vmax):
                kp = kp_ref[0:1, pl.ds(ip * T, T)]
                ccls = pbt_ref[0, 4:5, pl.ds(ip * T, T)]
                blk = _iou_block(cb_ref, pbt_ref, jc * T, ip * T, False)
                blk = jnp.where((kp > 0.0) & (rcls == ccls), blk, 0.0)
                return jnp.maximum(vmax, jnp.max(blk, axis=1, keepdims=True))

            vmax = jax.lax.fori_loop(0, NT, patch_body,
                                     jnp.zeros((T, 1), jnp.float32))
            sm_ref[1] += _rowdot(kc, vmax)[0, 0]

        return carry

    jax.lax.fori_loop(0, NT, clean_body, 0)
    cnt = jnp.sum(kc_ref[...])
    tot_ref[0] = jnp.full((1, 128), sm_ref[1], jnp.float32)
    cnt_ref[0] = jnp.full((1, 128), cnt, jnp.float32)


def _prep(xt, conf_thres):
    b = xt.shape[0]
    kfn = functools.partial(_prep_kernel, conf_thres=conf_thres)
    return pl.pallas_call(
        kfn,
        grid=(b,),
        in_specs=[pl.BlockSpec((1, 85, N), lambda i: (i, 0, 0))],
        out_specs=[
            pl.BlockSpec((1, 1, N), lambda i: (i, 0, 0)),
            pl.BlockSpec((1, 1, N), lambda i: (i, 0, 0)),
            pl.BlockSpec((1, 5, N), lambda i: (i, 0, 0)),
        ],
        out_shape=[
            jax.ShapeDtypeStruct((b, 1, N), jnp.float32),
            jax.ShapeDtypeStruct((b, 1, N), jnp.float32),
            jax.ShapeDtypeStruct((b, 5, N), jnp.float32),
        ],
    )(xt)


def _order_gather(conf, valid, boxt):
    key = jnp.where(valid[:, 0, :] > 0, -conf[:, 0, :], jnp.inf)
    order = jnp.argsort(key, axis=1, stable=True)
    bt = jnp.take_along_axis(boxt, order[:, None, :], axis=2)
    v = jnp.take_along_axis(valid, order[:, None, :], axis=2)
    return jnp.transpose(bt, (0, 2, 1)), bt, v


def kernel(output_clean, output_patch):
    x = jnp.concatenate([output_clean, output_patch], axis=0)
    x = jnp.pad(x, ((0, 0), (0, N - N_RAW), (0, 0)))
    xt = jnp.transpose(x, (0, 2, 1))
    b = output_clean.shape[0]

    cb, cbt, cv = _order_gather(*_prep(xt[:b], CONF_C))
    pb, pbt, pv = _order_gather(*_prep(xt[b:], CONF_P))

    spec_b = pl.BlockSpec((1, N, 5), lambda i: (i, 0, 0))
    spec_bt = pl.BlockSpec((1, 5, N), lambda i: (i, 0, 0))
    spec_v = pl.BlockSpec((1, 1, N), lambda i: (i, 0, 0))
    spec_o = pl.BlockSpec((1, 1, 128), lambda i: (i, 0, 0))
    tot, cnt = pl.pallas_call(
        _main_kernel,
        grid=(b,),
        in_specs=[spec_b, spec_bt, spec_v, spec_b, spec_bt, spec_v],
        out_specs=[spec_o, spec_o],
        out_shape=[
            jax.ShapeDtypeStruct((b, 1, 128), jnp.float32),
            jax.ShapeDtypeStruct((b, 1, 128), jnp.float32),
        ],
        scratch_shapes=[
            pltpu.VMEM((1, N), jnp.float32),
            pltpu.VMEM((1, N), jnp.float32),
            pltpu.SMEM((2,), jnp.float32),
        ],
    )(cb, cbt, cv, pb, pbt, pv)

    tsum = jnp.sum(tot[:, 0, 0])
    csum = jnp.sum(cnt[:, 0, 0])
    return jnp.where(csum > 0,
                     jnp.float32(1.0) - tsum / jnp.maximum(csum, 1.0),
                     jnp.float32(1.0))


# speedup vs baseline: 1.1425x; 1.0549x over previous
import functools

import jax
import jax.numpy as jnp
from jax.experimental import pallas as pl
from jax.experimental.pallas import tpu as pltpu

N_RAW = 5000
N = 5120
T = 512
NT = N // T
CONF_C = 0.25
CONF_P = 0.001
IOU_THR = 0.45
MAX_WH = 4096.0
CAP_C = 1000.0


def _prep_kernel(x_ref, conf_ref, valid_ref, boxt_ref, *, conf_thres):
    xt = x_ref[0]
    obj = xt[4:5, :]
    scores = xt[5:85, :] * obj
    conf = jnp.max(scores, axis=0, keepdims=True)
    ji = jax.lax.broadcasted_iota(jnp.int32, (80, N), 0).astype(jnp.float32)
    cls = jnp.min(jnp.where(scores == conf, ji, 1e9), axis=0, keepdims=True)
    valid = (obj > conf_thres) & (conf > conf_thres)
    xc, yc = xt[0:1, :], xt[1:2, :]
    w2, h2 = xt[2:3, :] * 0.5, xt[3:4, :] * 0.5
    x1 = xc - w2
    y1 = yc - h2
    x2 = xc + w2
    y2 = yc + h2
    conf_ref[0] = conf
    valid_ref[0] = valid.astype(jnp.float32)
    boxt_ref[0] = jnp.concatenate([x1, y1, x2, y2, cls], axis=0)


def _tile_coords(rb_ref, cbt_ref, r0, c0, offset):
    roff = rb_ref[0, pl.ds(r0, T), 4:5] * MAX_WH if offset else 0.0
    coff = cbt_ref[0, 4:5, pl.ds(c0, T)] * MAX_WH if offset else 0.0
    rx1 = rb_ref[0, pl.ds(r0, T), 0:1] + roff
    ry1 = rb_ref[0, pl.ds(r0, T), 1:2] + roff
    rx2 = rb_ref[0, pl.ds(r0, T), 2:3] + roff
    ry2 = rb_ref[0, pl.ds(r0, T), 3:4] + roff
    cx1 = cbt_ref[0, 0:1, pl.ds(c0, T)] + coff
    cy1 = cbt_ref[0, 1:2, pl.ds(c0, T)] + coff
    cx2 = cbt_ref[0, 2:3, pl.ds(c0, T)] + coff
    cy2 = cbt_ref[0, 3:4, pl.ds(c0, T)] + coff
    iw = jnp.maximum(jnp.minimum(rx2, cx2) - jnp.maximum(rx1, cx1), 0.0)
    ih = jnp.maximum(jnp.minimum(ry2, cy2) - jnp.maximum(ry1, cy1), 0.0)
    inter = iw * ih
    ra = (rx2 - rx1) * (ry2 - ry1)
    ca = (cx2 - cx1) * (cy2 - cy1)
    return inter, ra, ca


def _iou_block(rb_ref, cbt_ref, r0, c0, offset):
    inter, ra, ca = _tile_coords(rb_ref, cbt_ref, r0, c0, offset)
    return inter / (ra + ca - inter)


def _sup_block(rb_ref, cbt_ref, r0, c0):
    inter, ra, ca = _tile_coords(rb_ref, cbt_ref, r0, c0, True)
    denom = (ra + ca) - inter
    return ((inter > IOU_THR * denom) | (denom == 0.0)).astype(jnp.float32)


def _rowdot(k, m):
    return jnp.dot(k, m, preferred_element_type=jnp.float32)


def _nms(b_ref, bt_ref, v_ref, keep_ref, sm_ref, cap):
    keep_ref[...] = jnp.zeros((1, N), jnp.float32)
    sm_ref[0] = 0.0

    def tile_body(t, carry):
        c0 = t * T

        @pl.when((sm_ref[0] < cap) if cap is not None else (t >= 0))
        def _():
            v = v_ref[0, 0:1, pl.ds(c0, T)]

            def cross_body(u, sup):
                ku = keep_ref[0:1, pl.ds(u * T, T)]
                return sup + _rowdot(ku, _sup_block(b_ref, bt_ref, u * T, c0))

            sup = jax.lax.fori_loop(0, t, cross_body,
                                    jnp.zeros((1, T), jnp.float32))
            v2 = jnp.where(sup == 0.0, v, 0.0)

            ii = jax.lax.broadcasted_iota(jnp.int32, (T, T), 0)
            jj = jax.lax.broadcasted_iota(jnp.int32, (T, T), 1)
            s_mat = jnp.where(ii < jj, _sup_block(b_ref, bt_ref, c0, c0), 0.0)

            def cond(st):
                return st[1]

            def body(st):
                k, _ = st
                k2 = jnp.where(_rowdot(k, s_mat) == 0.0, v2, 0.0)
                return k2, jnp.any(k2 != k)

            k, _ = jax.lax.while_loop(cond, body, (v2, jnp.bool_(True)))
            if cap is not None:
                u_incl = (ii <= jj).astype(jnp.float32)
                pref = _rowdot(k, u_incl)
                k = jnp.where(sm_ref[0] + pref <= cap, k, 0.0)
            keep_ref[0:1, pl.ds(c0, T)] = k
            sm_ref[0] += jnp.sum(k)

        return carry

    jax.lax.fori_loop(0, NT, tile_body, 0)


def _main_kernel(cb_ref, cbt_ref, cv_ref, pb_ref, pbt_ref, pv_ref,
                 tot_ref, cnt_ref, kc_ref, kp_ref, sm_ref):
    _nms(cb_ref, cbt_ref, cv_ref, kc_ref, sm_ref, CAP_C)
    _nms(pb_ref, pbt_ref, pv_ref, kp_ref, sm_ref, None)

    sm_ref[1] = 0.0

    def clean_body(jc, carry):
        kc = kc_ref[0:1, pl.ds(jc * T, T)]

        @pl.when(jnp.sum(kc) > 0.0)
        def _():
            rcls = cb_ref[0, pl.ds(jc * T, T), 4:5]

            def patch_body(ip, vmax):
                kp = kp_ref[0:1, pl.ds(ip * T, T)]
                ccls = pbt_ref[0, 4:5, pl.ds(ip * T, T)]
                blk = _iou_block(cb_ref, pbt_ref, jc * T, ip * T, False)
                blk = jnp.where((kp > 0.0) & (rcls == ccls), blk, 0.0)
                return jnp.maximum(vmax, jnp.max(blk, axis=1, keepdims=True))

            vmax = jax.lax.fori_loop(0, NT, patch_body,
                                     jnp.zeros((T, 1), jnp.float32))
            sm_ref[1] += _rowdot(kc, vmax)[0, 0]

        return carry

    jax.lax.fori_loop(0, NT, clean_body, 0)
    cnt = jnp.sum(kc_ref[...])
    tot_ref[0] = jnp.full((1, 128), sm_ref[1], jnp.float32)
    cnt_ref[0] = jnp.full((1, 128), cnt, jnp.float32)


def _prep(xt, conf_thres):
    b = xt.shape[0]
    kfn = functools.partial(_prep_kernel, conf_thres=conf_thres)
    return pl.pallas_call(
        kfn,
        grid=(b,),
        in_specs=[pl.BlockSpec((1, 85, N), lambda i: (i, 0, 0))],
        out_specs=[
            pl.BlockSpec((1, 1, N), lambda i: (i, 0, 0)),
            pl.BlockSpec((1, 1, N), lambda i: (i, 0, 0)),
            pl.BlockSpec((1, 5, N), lambda i: (i, 0, 0)),
        ],
        out_shape=[
            jax.ShapeDtypeStruct((b, 1, N), jnp.float32),
            jax.ShapeDtypeStruct((b, 1, N), jnp.float32),
            jax.ShapeDtypeStruct((b, 5, N), jnp.float32),
        ],
    )(xt)


def _order_gather(conf, valid, boxt):
    key = jnp.where(valid[:, 0, :] > 0, -conf[:, 0, :], jnp.inf)
    order = jnp.argsort(key, axis=1, stable=True)
    bt = jnp.take_along_axis(boxt, order[:, None, :], axis=2)
    v = jnp.take_along_axis(valid, order[:, None, :], axis=2)
    return jnp.transpose(bt, (0, 2, 1)), bt, v


def kernel(output_clean, output_patch):
    x = jnp.concatenate([output_clean, output_patch], axis=0)
    x = jnp.pad(x, ((0, 0), (0, N - N_RAW), (0, 0)))
    xt = jnp.transpose(x, (0, 2, 1))
    b = output_clean.shape[0]

    cb, cbt, cv = _order_gather(*_prep(xt[:b], CONF_C))
    pb, pbt, pv = _order_gather(*_prep(xt[b:], CONF_P))

    spec_b = pl.BlockSpec((1, N, 5), lambda i: (i, 0, 0))
    spec_bt = pl.BlockSpec((1, 5, N), lambda i: (i, 0, 0))
    spec_v = pl.BlockSpec((1, 1, N), lambda i: (i, 0, 0))
    spec_o = pl.BlockSpec((1, 1, 128), lambda i: (i, 0, 0))
    tot, cnt = pl.pallas_call(
        _main_kernel,
        grid=(b,),
        in_specs=[spec_b, spec_bt, spec_v, spec_b, spec_bt, spec_v],
        out_specs=[spec_o, spec_o],
        out_shape=[
            jax.ShapeDtypeStruct((b, 1, 128), jnp.float32),
            jax.ShapeDtypeStruct((b, 1, 128), jnp.float32),
        ],
        scratch_shapes=[
            pltpu.VMEM((1, N), jnp.float32),
            pltpu.VMEM((1, N), jnp.float32),
            pltpu.SMEM((2,), jnp.float32),
        ],
    )(cb, cbt, cv, pb, pbt, pv)

    tsum = jnp.sum(tot[:, 0, 0])
    csum = jnp.sum(cnt[:, 0, 0])
    return jnp.where(csum > 0,
                     jnp.float32(1.0) - tsum / jnp.maximum(csum, 1.0),
                     jnp.float32(1.0))
